# Initial kernel scaffold; baseline (speedup 1.0000x reference)
import jax
import jax.numpy as jnp
from jax.experimental import pallas as pl


def kernel(indices, offsets, weights):
    raise NotImplementedError("write your pallas kernel here")


# speedup vs baseline: 1.2553x; 1.2553x over previous
import functools

import jax
import jax.numpy as jnp
from jax import lax
from jax.experimental import pallas as pl
from jax.experimental.pallas import tpu as pltpu
from jax.experimental.pallas import tpu_sc as plsc

_N_TABLES = 26
_BATCH = 16384
_DIM = 16
_B = _N_TABLES * _BATCH

_NC = 2
_NS = 16
_NW = _NC * _NS
_G = 128
_ROWS = _B // _G
_ROWS_PER_W = _ROWS // _NW
_K = 8
_NCHUNKS = _ROWS_PER_W // _K
_NBUF = 2


@functools.partial(
    pl.kernel,
    mesh=plsc.VectorSubcoreMesh(core_axis_name="c", subcore_axis_name="s"),
    out_type=jax.ShapeDtypeStruct((_ROWS, _G, _DIM), jnp.float32),
    scratch_types=[
        pltpu.VMEM((_NBUF, _K, _G), jnp.int32),
        pltpu.VMEM((_NBUF, _K, _G, _DIM), jnp.float32),
        pltpu.SemaphoreType.DMA,
        pltpu.SemaphoreType.DMA,
    ],
    compiler_params=pltpu.CompilerParams(use_tc_tiling_on_sc=False),
)
def _gather(table_hbm, idx_hbm, out_hbm, idx_v, rows_v, gsem0, gsem1):
    wid = lax.axis_index("s") * _NC + lax.axis_index("c")
    base = wid * _ROWS_PER_W
    gsems = (gsem0, gsem1)

    def fire(buf, row0):
        pltpu.sync_copy(idx_hbm.at[pl.ds(row0, _K)], idx_v.at[buf])
        copies = []
        for j in range(_K):
            copies.append(pltpu.async_copy(
                table_hbm.at[idx_v.at[buf, j]], rows_v.at[buf, j],
                gsems[buf]))
        return copies

    inflight = [None] * _NBUF
    for b in range(min(_NBUF - 1, _NCHUNKS)):
        inflight[b] = fire(b, base + b * _K)

    for g in range(_NCHUNKS):
        cur = g % _NBUF
        nxt = g + _NBUF - 1
        if nxt < _NCHUNKS:
            inflight[nxt % _NBUF] = fire(nxt % _NBUF, base + nxt * _K)
        for c in inflight[cur]:
            c.wait()
        pltpu.sync_copy(rows_v.at[cur],
                        out_hbm.at[pl.ds(base + g * _K, _K)])


def kernel(indices, offsets, weights):
    del offsets
    idx2d = indices.astype(jnp.int32).reshape(_ROWS, _G)
    out = _gather(weights, idx2d)
    return out.reshape(_N_TABLES, _BATCH, _DIM)


# speedup vs baseline: 1.2622x; 1.0055x over previous
import functools

import jax
import jax.numpy as jnp
from jax import lax
from jax.experimental import pallas as pl
from jax.experimental.pallas import tpu as pltpu
from jax.experimental.pallas import tpu_sc as plsc

_N_TABLES = 26
_BATCH = 16384
_DIM = 16
_B = _N_TABLES * _BATCH

_NC = 2
_NS = 16
_NW = _NC * _NS
_G = 128
_ROWS = _B // _G
_ROWS_PER_W = _ROWS // _NW
_K = 13
_NCHUNKS = _ROWS_PER_W // _K
_NBUF = 4


@functools.partial(
    pl.kernel,
    mesh=plsc.VectorSubcoreMesh(core_axis_name="c", subcore_axis_name="s"),
    out_type=jax.ShapeDtypeStruct((_ROWS, _G, _DIM), jnp.float32),
    scratch_types=[
        pltpu.VMEM((_NBUF, _K, _G), jnp.int32),
        pltpu.VMEM((_NBUF, _K, _G, _DIM), jnp.float32),
        pltpu.SemaphoreType.DMA((_NBUF,)),
    ],
    compiler_params=pltpu.CompilerParams(use_tc_tiling_on_sc=False),
)
def _gather(table_hbm, idx_hbm, out_hbm, idx_v, rows_v, gsems):
    wid = lax.axis_index("s") * _NC + lax.axis_index("c")
    base = wid * _ROWS_PER_W

    def fire(buf, row0):
        pltpu.sync_copy(idx_hbm.at[pl.ds(row0, _K)], idx_v.at[buf])
        copies = []
        for j in range(_K):
            copies.append(pltpu.async_copy(
                table_hbm.at[idx_v.at[buf, j]], rows_v.at[buf, j],
                gsems.at[buf]))
        return copies

    inflight = [None] * _NBUF
    for b in range(min(_NBUF - 1, _NCHUNKS)):
        inflight[b] = fire(b, base + b * _K)

    for g in range(_NCHUNKS):
        cur = g % _NBUF
        nxt = g + _NBUF - 1
        if nxt < _NCHUNKS:
            inflight[nxt % _NBUF] = fire(nxt % _NBUF, base + nxt * _K)
        for c in inflight[cur]:
            c.wait()
        pltpu.sync_copy(rows_v.at[cur],
                        out_hbm.at[pl.ds(base + g * _K, _K)])


def kernel(indices, offsets, weights):
    del offsets
    idx2d = indices.astype(jnp.int32).reshape(_ROWS, _G)
    out = _gather(weights, idx2d)
    return out.reshape(_N_TABLES, _BATCH, _DIM)


# speedup vs baseline: 1.3333x; 1.0563x over previous
import functools

import jax
import jax.numpy as jnp
from jax import lax
from jax.experimental import pallas as pl
from jax.experimental.pallas import tpu as pltpu
from jax.experimental.pallas import tpu_sc as plsc

_N_TABLES = 26
_BATCH = 16384
_DIM = 16
_B = _N_TABLES * _BATCH
_V = _N_TABLES * 100000

_NC = 2
_NS = 16
_NW = _NC * _NS

_FULL_TILES = _V // 128
_WP_ROWS = _V * _DIM // 128
_TILES_LO = _FULL_TILES // _NW
_TILES_EXTRA = _FULL_TILES - _TILES_LO * _NW

_G = 128
_ROWS = _B // _G
_ROWS_PER_W = _ROWS // _NW
_CBLK = _BATCH // _G


@functools.partial(
    pl.kernel,
    mesh=plsc.VectorSubcoreMesh(core_axis_name="c", subcore_axis_name="s"),
    out_type=jax.ShapeDtypeStruct((_WP_ROWS, 128), jnp.float32),
    scratch_types=[
        pltpu.VMEM((2, 16, 128), jnp.float32),
        pltpu.VMEM((2, 16, 128), jnp.float32),
        pltpu.VMEM((64, 16), jnp.float32),
        pltpu.SemaphoreType.DMA((2,)),
    ],
    compiler_params=pltpu.CompilerParams(use_tc_tiling_on_sc=True, needs_layout_passes=False),
)
def _pack(wt_hbm, tail_hbm, wp_hbm, inb, outb, tailb, isems):
    wid = lax.axis_index("s") * _NC + lax.axis_index("c")
    ntiles = jnp.where(wid < _TILES_EXTRA, _TILES_LO + 1, _TILES_LO)
    tile0 = wid * _TILES_LO + jnp.minimum(wid, _TILES_EXTRA)

    def fire(t, buf):
        pltpu.async_copy(
            wt_hbm.at[:, pl.ds(pl.multiple_of(t * 128, 128), 128)],
            inb.at[buf], isems.at[buf])

    def wait(buf):
        pltpu.make_async_copy(
            wt_hbm.at[:, pl.ds(0, 128)], inb.at[buf], isems.at[buf]).wait()

    def transpose_tile(buf):
        def col(j, _):
            vec = plsc.load_gather(
                inb.at[buf],
                [lax.iota(jnp.int32, 16), jnp.full((16,), j, jnp.int32)])
            outb[buf, j // 8, pl.ds((j % 8) * 16, 16)] = vec
            return ()
        lax.fori_loop(0, 128, col, (), unroll=8)

    fire(tile0, 0)

    def body(i, _):
        cur = i % 2
        nxt = (i + 1) % 2

        @pl.when(i + 1 < ntiles)
        def _():
            fire(tile0 + i + 1, nxt)

        wait(cur)
        transpose_tile(cur)
        pltpu.sync_copy(outb.at[cur],
                        wp_hbm.at[pl.ds((tile0 + i) * 16, 16)])
        return ()

    lax.fori_loop(0, ntiles, body, ())

    @pl.when(wid == 0)
    def _():
        pltpu.sync_copy(tail_hbm, tailb)
        for k in range(64):
            outb[0, 0, pl.ds((k % 8) * 16, 16)] = tailb[k, :]
            if k % 8 == 7:
                pltpu.sync_copy(outb.at[0, 0],
                                wp_hbm.at[_FULL_TILES * 16 + k // 8])


@functools.partial(
    pl.kernel,
    mesh=plsc.VectorSubcoreMesh(core_axis_name="c", subcore_axis_name="s"),
    out_type=jax.ShapeDtypeStruct((_N_TABLES, _DIM, _BATCH), jnp.float32),
    scratch_types=[
        pltpu.VMEM((2, 128), jnp.int32),
        pltpu.VMEM((2, 128), jnp.int32),
        pltpu.VMEM((2, 128, 128), jnp.float32),
        pltpu.VMEM((2, 16, 128), jnp.float32),
        pltpu.SemaphoreType.DMA((2,)),
    ],
    compiler_params=pltpu.CompilerParams(use_tc_tiling_on_sc=True, needs_layout_passes=False),
)
def _gather(wp_hbm, idx_hbm, out_hbm, idxraw, idxrow, rowsv, outv, gsems):
    wid = lax.axis_index("s") * _NC + lax.axis_index("c")
    base = wid * _ROWS_PER_W

    def fire(blk, buf):
        pltpu.sync_copy(idx_hbm.at[blk], idxraw.at[buf])
        for kk in range(8):
            sl = pl.ds(kk * 16, 16)
            idxrow[buf, sl] = lax.shift_right_logical(idxraw[buf, sl], 3)
        pltpu.async_copy(wp_hbm.at[idxrow.at[buf]], rowsv.at[buf],
                         gsems.at[buf])

    def wait(buf):
        pltpu.make_async_copy(
            wp_hbm.at[idxrow.at[buf]], rowsv.at[buf], gsems.at[buf]).wait()

    def drain(blk, buf):
        t = blk // _CBLK
        c = blk % _CBLK
        wait(buf)

        rows2d = rowsv.at[buf]
        for j0 in range(0, 128, 16):
            vv = idxraw[buf, pl.ds(j0, 16)]
            colbase = (vv & 7) * 16
            rowix = lax.iota(jnp.int32, 16) + j0
            for d in range(_DIM):
                vec = plsc.load_gather(rows2d, [rowix, colbase + d])
                outv[buf, d, pl.ds(j0, 16)] = vec
        pltpu.sync_copy(
            outv.at[buf],
            out_hbm.at[t, :, pl.ds(pl.multiple_of(c * 128, 128), 128)])

    fire(base, 0)

    def body(i, _):
        cur = i % 2
        nxt = (i + 1) % 2

        @pl.when(i + 1 < _ROWS_PER_W)
        def _():
            fire(base + i + 1, nxt)

        drain(base + i, cur)
        return ()

    lax.fori_loop(0, _ROWS_PER_W, body, ())


def kernel(indices, offsets, weights):
    del offsets
    wt = weights.T
    tail = weights[_FULL_TILES * 128:, :]
    wp = _pack(wt, tail)
    idx2d = indices.astype(jnp.int32).reshape(_ROWS, _G)
    out = _gather(wp, idx2d)
    return out.transpose(0, 2, 1)
